# baseline (speedup 1.0000x reference)
import functools

import jax
import jax.numpy as jnp
from jax import lax
from jax.experimental import pallas as pl
from jax.experimental.pallas import tpu as pltpu
from jax.experimental.pallas import tpu_sc as plsc

NUM_ROWS = 2600000
BATCH = 16384
N_FIELDS = 26

NC = 2
NS = 16
L = 16

HALF = NUM_ROWS // NC
ROWS_W = BATCH // NS
PIECE_F = 2
NPIECE = N_FIELDS // PIECE_F
PIECE = PIECE_F * ROWS_W

STAGE = 81280
STAGE_LAST = HALF - (NS - 1) * STAGE
SB = 10160
SB_LAST = 8080

_mesh = plsc.VectorSubcoreMesh(core_axis_name="c", subcore_axis_name="s")


@functools.partial(
    pl.kernel,
    out_type=jax.ShapeDtypeStruct((NC, BATCH), jnp.float32),
    mesh=_mesh,
    scratch_types=[
        pltpu.VMEM_SHARED((HALF + L,), jnp.float32),
        pltpu.VMEM((PIECE,), jnp.int32),
        pltpu.VMEM((PIECE,), jnp.int32),
        pltpu.VMEM((PIECE,), jnp.float32),
        pltpu.VMEM((PIECE,), jnp.float32),
        pltpu.VMEM((SB,), jnp.float32),
        pltpu.VMEM((SB,), jnp.float32),
        pltpu.VMEM((ROWS_W,), jnp.float32),
        pltpu.VMEM((L,), jnp.float32),
        pltpu.SemaphoreType.DMA,
        pltpu.SemaphoreType.DMA,
        pltpu.SemaphoreType.DMA,
        pltpu.SemaphoreType.DMA,
    ],
)
def _sc_embed_sum(idx_hbm, table_hbm, bias_hbm, out_hbm,
                  shared_v, idx0, idx1, val0, val1, sbuf0, sbuf1,
                  out_v, bias_v,
                  in_sem, out_sem, idx_sem, gsem):
    c = lax.axis_index("c")
    s = lax.axis_index("s")
    idxb = (idx0, idx1)
    valb = (val0, val1)
    sbufs = (sbuf0, sbuf1)

    idx_d = pltpu.async_copy(idx_hbm.at[s * NPIECE], idx0, idx_sem)

    pltpu.sync_copy(bias_hbm, bias_v)

    def _run_stage(nchunks, sz):
        hb = c * HALF + s * STAGE
        sb = s * STAGE
        pend = [pltpu.async_copy(table_hbm.at[pl.ds(hb, sz)],
                                 sbufs[0].at[pl.ds(0, sz)], in_sem)]
        for k in range(nchunks):
            if k >= 2:
                pltpu.make_async_copy(sbufs[k % 2].at[pl.ds(0, sz)],
                                      shared_v.at[pl.ds(sb + (k - 2) * sz, sz)],
                                      out_sem).wait()
            if k + 1 < nchunks:
                pend.append(pltpu.async_copy(
                    table_hbm.at[pl.ds(hb + (k + 1) * sz, sz)],
                    sbufs[(k + 1) % 2].at[pl.ds(0, sz)], in_sem))
            pend[k].wait()
            pltpu.async_copy(sbufs[k % 2].at[pl.ds(0, sz)],
                             shared_v.at[pl.ds(sb + k * sz, sz)], out_sem)
        for k in range(nchunks - 2, nchunks):
            pltpu.make_async_copy(sbufs[k % 2].at[pl.ds(0, sz)],
                                  shared_v.at[pl.ds(sb + k * sz, sz)],
                                  out_sem).wait()

    @pl.when(s < NS - 1)
    def _stage_most():
        _run_stage(STAGE // SB, SB)

    @pl.when(s == NS - 1)
    def _stage_last():
        _run_stage(STAGE_LAST // SB_LAST, SB_LAST)

    @pl.when(s == 0)
    def _zero_slot():
        out_v[pl.ds(0, L)] = jnp.zeros((L,), jnp.float32)
        pltpu.sync_copy(out_v.at[pl.ds(0, L)], shared_v.at[pl.ds(HALF, L)])

    plsc.subcore_barrier()

    bias_vec = bias_v[...] * (1 - c).astype(jnp.float32)

    @pl.loop(0, ROWS_W // L)
    def _init(j):
        out_v[pl.ds(j * L, L)] = bias_vec

    lo = c * HALF

    def _remap(buf):
        @pl.loop(0, PIECE // L)
        def _r(i):
            v = buf[pl.ds(i * L, L)]
            local = v - lo
            owned = (v >= lo) & (local < HALF)
            buf[pl.ds(i * L, L)] = jnp.where(owned, local, HALF)

    def _reduce(buf):
        @pl.loop(0, ROWS_W // L)
        def _g(j):
            acc = out_v[pl.ds(j * L, L)]
            for f in range(PIECE_F):
                acc = acc + buf[pl.ds(f * ROWS_W + j * L, L)]
            out_v[pl.ds(j * L, L)] = acc

    gather_d = [None] * NPIECE
    for p in range(NPIECE):
        idx_d.wait()
        _remap(idxb[p % 2])
        gather_d[p] = pltpu.async_copy(shared_v.at[idxb[p % 2]],
                                       valb[p % 2], gsem)
        if p >= 1:
            gather_d[p - 1].wait()
        if p + 1 < NPIECE:
            idx_d = pltpu.async_copy(idx_hbm.at[s * NPIECE + p + 1],
                                     idxb[(p + 1) % 2], idx_sem)
        if p >= 1:
            _reduce(valb[(p - 1) % 2])
    gather_d[NPIECE - 1].wait()
    _reduce(valb[(NPIECE - 1) % 2])

    pltpu.sync_copy(out_v, out_hbm.at[c, pl.ds(s * ROWS_W, ROWS_W)])


def kernel(x, fc_weight, bias):
    idx = (x.astype(jnp.int32)
           .T.reshape(NPIECE, PIECE_F, NS, ROWS_W)
           .transpose(2, 0, 1, 3)
           .reshape(NS * NPIECE, PIECE))
    table = fc_weight.reshape(NUM_ROWS)
    bias_b = jnp.broadcast_to(bias.astype(jnp.float32), (L,))
    parts = _sc_embed_sum(idx, table, bias_b)
    return (parts[0] + parts[1]).reshape(BATCH, 1)


# speedup vs baseline: 1.5566x; 1.5566x over previous
import functools

import jax
import jax.numpy as jnp
from jax import lax
from jax.experimental import pallas as pl
from jax.experimental.pallas import tpu as pltpu
from jax.experimental.pallas import tpu_sc as plsc

NUM_ROWS = 2600000
BATCH = 16384
N_FIELDS = 26

NC = 2
NS = 16
L = 16

HALF = NUM_ROWS // NC
ROWS_W = BATCH // NS
PIECE_F = 2
NPIECE = N_FIELDS // PIECE_F
PIECE = PIECE_F * ROWS_W

STAGE = 81280
STAGE_LAST = HALF - (NS - 1) * STAGE
SB = 10160
SB_LAST = 8080

_mesh = plsc.VectorSubcoreMesh(core_axis_name="c", subcore_axis_name="s")

_SQ_BLK = 131072
_SQ_GRID = -(-NUM_ROWS // _SQ_BLK)


def _tc_squeeze_body(w_ref, o_ref):
    o_ref[...] = w_ref[0, :]


_tc_squeeze = pl.pallas_call(
    _tc_squeeze_body,
    out_shape=jax.ShapeDtypeStruct((NUM_ROWS,), jnp.float32),
    grid=(_SQ_GRID,),
    in_specs=[pl.BlockSpec((1, _SQ_BLK), lambda i: (0, i))],
    out_specs=pl.BlockSpec((_SQ_BLK,), lambda i: (i,)),
)


@functools.partial(
    pl.kernel,
    out_type=jax.ShapeDtypeStruct((NC, BATCH), jnp.float32),
    mesh=_mesh,
    scratch_types=[
        pltpu.VMEM_SHARED((HALF + L,), jnp.float32),
        pltpu.VMEM((PIECE,), jnp.int32),
        pltpu.VMEM((PIECE,), jnp.int32),
        pltpu.VMEM((PIECE,), jnp.float32),
        pltpu.VMEM((PIECE,), jnp.float32),
        pltpu.VMEM((SB,), jnp.float32),
        pltpu.VMEM((SB,), jnp.float32),
        pltpu.VMEM((ROWS_W,), jnp.float32),
        pltpu.VMEM((L,), jnp.float32),
        pltpu.SemaphoreType.DMA,
        pltpu.SemaphoreType.DMA,
        pltpu.SemaphoreType.DMA,
        pltpu.SemaphoreType.DMA,
    ],
)
def _sc_embed_sum(idx_hbm, table_hbm, bias_hbm, out_hbm,
                  shared_v, idx0, idx1, val0, val1, sbuf0, sbuf1,
                  out_v, bias_v,
                  in_sem, out_sem, idx_sem, gsem):
    c = lax.axis_index("c")
    s = lax.axis_index("s")
    idxb = (idx0, idx1)
    valb = (val0, val1)
    sbufs = (sbuf0, sbuf1)

    idx_d = pltpu.async_copy(idx_hbm.at[s * NPIECE], idx0, idx_sem)

    pltpu.sync_copy(bias_hbm, bias_v)

    def _run_stage(nchunks, sz):
        hb = c * HALF + s * STAGE
        sb = s * STAGE
        pend = [pltpu.async_copy(table_hbm.at[pl.ds(hb, sz)],
                                 sbufs[0].at[pl.ds(0, sz)], in_sem)]
        for k in range(nchunks):
            if k >= 2:
                pltpu.make_async_copy(sbufs[k % 2].at[pl.ds(0, sz)],
                                      shared_v.at[pl.ds(sb + (k - 2) * sz, sz)],
                                      out_sem).wait()
            if k + 1 < nchunks:
                pend.append(pltpu.async_copy(
                    table_hbm.at[pl.ds(hb + (k + 1) * sz, sz)],
                    sbufs[(k + 1) % 2].at[pl.ds(0, sz)], in_sem))
            pend[k].wait()
            pltpu.async_copy(sbufs[k % 2].at[pl.ds(0, sz)],
                             shared_v.at[pl.ds(sb + k * sz, sz)], out_sem)
        for k in range(nchunks - 2, nchunks):
            pltpu.make_async_copy(sbufs[k % 2].at[pl.ds(0, sz)],
                                  shared_v.at[pl.ds(sb + k * sz, sz)],
                                  out_sem).wait()

    @pl.when(s < NS - 1)
    def _stage_most():
        _run_stage(STAGE // SB, SB)

    @pl.when(s == NS - 1)
    def _stage_last():
        _run_stage(STAGE_LAST // SB_LAST, SB_LAST)

    @pl.when(s == 0)
    def _zero_slot():
        out_v[pl.ds(0, L)] = jnp.zeros((L,), jnp.float32)
        pltpu.sync_copy(out_v.at[pl.ds(0, L)], shared_v.at[pl.ds(HALF, L)])

    plsc.subcore_barrier()

    bias_vec = bias_v[...] * (1 - c).astype(jnp.float32)

    @pl.loop(0, ROWS_W // L)
    def _init(j):
        out_v[pl.ds(j * L, L)] = bias_vec

    lo = c * HALF

    def _remap(buf):
        @pl.loop(0, PIECE // L)
        def _r(i):
            v = buf[pl.ds(i * L, L)]
            local = v - lo
            owned = (v >= lo) & (local < HALF)
            buf[pl.ds(i * L, L)] = jnp.where(owned, local, HALF)

    def _reduce(buf):
        @pl.loop(0, ROWS_W // L)
        def _g(j):
            acc = out_v[pl.ds(j * L, L)]
            for f in range(PIECE_F):
                acc = acc + buf[pl.ds(f * ROWS_W + j * L, L)]
            out_v[pl.ds(j * L, L)] = acc

    gather_d = [None] * NPIECE
    for p in range(NPIECE):
        idx_d.wait()
        _remap(idxb[p % 2])
        gather_d[p] = pltpu.async_copy(shared_v.at[idxb[p % 2]],
                                       valb[p % 2], gsem)
        if p >= 1:
            gather_d[p - 1].wait()
        if p + 1 < NPIECE:
            idx_d = pltpu.async_copy(idx_hbm.at[s * NPIECE + p + 1],
                                     idxb[(p + 1) % 2], idx_sem)
        if p >= 1:
            _reduce(valb[(p - 1) % 2])
    gather_d[NPIECE - 1].wait()
    _reduce(valb[(NPIECE - 1) % 2])

    pltpu.sync_copy(out_v, out_hbm.at[c, pl.ds(s * ROWS_W, ROWS_W)])


def kernel(x, fc_weight, bias):
    idx = (x.astype(jnp.int32)
           .T.reshape(NPIECE, PIECE_F, NS, ROWS_W)
           .transpose(2, 0, 1, 3)
           .reshape(NS * NPIECE, PIECE))
    table = _tc_squeeze(fc_weight.T)
    bias_b = jnp.broadcast_to(bias.astype(jnp.float32), (L,))
    parts = _sc_embed_sum(idx, table, bias_b)
    return (parts[0] + parts[1]).reshape(BATCH, 1)


# speedup vs baseline: 4.7156x; 3.0294x over previous
import functools

import jax
import jax.numpy as jnp
from jax import lax
from jax.experimental import pallas as pl
from jax.experimental.pallas import tpu as pltpu
from jax.experimental.pallas import tpu_sc as plsc

NUM_ROWS = 2600000
BATCH = 16384
N_FIELDS = 26

NC = 2
NS = 16
L = 16
NW = NC * NS
ROWS_W = BATCH // NW
IDX_W = ROWS_W * N_FIELDS
CHUNK = 128
NCHUNK = IDX_W // CHUNK

_mesh = plsc.VectorSubcoreMesh(core_axis_name="c", subcore_axis_name="s")

_SQ_BLK = 131072
_SQ_GRID = -(-NUM_ROWS // _SQ_BLK)


def _tc_squeeze_body(w_ref, o_ref):
    o_ref[...] = w_ref[0, :]


_tc_squeeze = pl.pallas_call(
    _tc_squeeze_body,
    out_shape=jax.ShapeDtypeStruct((NUM_ROWS,), jnp.float32),
    grid=(_SQ_GRID,),
    in_specs=[pl.BlockSpec((1, _SQ_BLK), lambda i: (0, i))],
    out_specs=pl.BlockSpec((_SQ_BLK,), lambda i: (i,)),
)


@functools.partial(
    pl.kernel,
    out_type=jax.ShapeDtypeStruct((BATCH,), jnp.float32),
    mesh=_mesh,
    scratch_types=[
        pltpu.VMEM((IDX_W,), jnp.int32),
        pltpu.VMEM((IDX_W,), jnp.float32),
        pltpu.VMEM((ROWS_W,), jnp.float32),
        pltpu.VMEM((L,), jnp.float32),
        pltpu.SemaphoreType.DMA,
    ],
)
def _sc_embed_sum(idx_hbm, table_hbm, bias_hbm, out_hbm,
                  idx_v, vals_v, out_v, bias_v, sem):
    wid = lax.axis_index("s") * NC + lax.axis_index("c")
    base = wid * ROWS_W

    pltpu.sync_copy(bias_hbm, bias_v)
    pltpu.sync_copy(idx_hbm.at[wid], idx_v)

    @pl.loop(0, NCHUNK)
    def _fire(ch):
        pltpu.async_copy(table_hbm.at[idx_v.at[pl.ds(ch * CHUNK, CHUNK)]],
                         vals_v.at[pl.ds(ch * CHUNK, CHUNK)], sem)

    pltpu.make_async_copy(table_hbm.at[pl.ds(0, IDX_W)], vals_v, sem).wait()

    bias_vec = bias_v[...]

    @pl.loop(0, ROWS_W // L)
    def _reduce(j):
        acc = bias_vec
        for f in range(N_FIELDS):
            acc = acc + vals_v[pl.ds(f * ROWS_W + j * L, L)]
        out_v[pl.ds(j * L, L)] = acc

    pltpu.sync_copy(out_v, out_hbm.at[pl.ds(base, ROWS_W)])


def kernel(x, fc_weight, bias):
    idx = (x.astype(jnp.int32)
           .T.reshape(N_FIELDS, NW, ROWS_W)
           .transpose(1, 0, 2)
           .reshape(NW, IDX_W))
    table = _tc_squeeze(fc_weight.T)
    bias_b = jnp.broadcast_to(bias.astype(jnp.float32), (L,))
    out = _sc_embed_sum(idx, table, bias_b)
    return out.reshape(BATCH, 1)


# speedup vs baseline: 5.2742x; 1.1184x over previous
import functools

import jax
import jax.numpy as jnp
from jax import lax
from jax.experimental import pallas as pl
from jax.experimental.pallas import tpu as pltpu
from jax.experimental.pallas import tpu_sc as plsc

NUM_ROWS = 2600000
BATCH = 16384
N_FIELDS = 26

NC = 2
NS = 16
L = 16
NW = NC * NS
ROWS_W = BATCH // NW
IDX_W = ROWS_W * N_FIELDS
CHUNK = 128
NCHUNK = IDX_W // CHUNK

_mesh = plsc.VectorSubcoreMesh(core_axis_name="c", subcore_axis_name="s")

_SQ_BLK = 524288
_SQ_GRID = -(-NUM_ROWS // _SQ_BLK)


def _tc_squeeze_body(w_ref, o_ref):
    o_ref[...] = w_ref[0, :]


_tc_squeeze = pl.pallas_call(
    _tc_squeeze_body,
    out_shape=jax.ShapeDtypeStruct((NUM_ROWS,), jnp.float32),
    grid=(_SQ_GRID,),
    in_specs=[pl.BlockSpec((1, _SQ_BLK), lambda i: (0, i))],
    out_specs=pl.BlockSpec((_SQ_BLK,), lambda i: (i,)),
)


@functools.partial(
    pl.kernel,
    out_type=jax.ShapeDtypeStruct((BATCH,), jnp.float32),
    mesh=_mesh,
    scratch_types=[
        pltpu.VMEM((IDX_W,), jnp.int32),
        pltpu.VMEM((IDX_W,), jnp.float32),
        pltpu.VMEM((ROWS_W,), jnp.float32),
        pltpu.VMEM((L,), jnp.float32),
        pltpu.SemaphoreType.DMA,
    ],
)
def _sc_embed_sum(idx_hbm, table_hbm, bias_hbm, out_hbm,
                  idx_v, vals_v, out_v, bias_v, sem):
    wid = lax.axis_index("s") * NC + lax.axis_index("c")
    base = wid * ROWS_W

    pltpu.sync_copy(bias_hbm, bias_v)
    pltpu.sync_copy(idx_hbm.at[wid], idx_v)

    @pl.loop(0, NCHUNK)
    def _fire(ch):
        pltpu.async_copy(table_hbm.at[idx_v.at[pl.ds(ch * CHUNK, CHUNK)]],
                         vals_v.at[pl.ds(ch * CHUNK, CHUNK)], sem)

    pltpu.make_async_copy(table_hbm.at[pl.ds(0, IDX_W)], vals_v, sem).wait()

    bias_vec = bias_v[...]

    @pl.loop(0, ROWS_W // L)
    def _reduce(j):
        acc = bias_vec
        for f in range(N_FIELDS):
            acc = acc + vals_v[pl.ds(f * ROWS_W + j * L, L)]
        out_v[pl.ds(j * L, L)] = acc

    pltpu.sync_copy(out_v, out_hbm.at[pl.ds(base, ROWS_W)])


def kernel(x, fc_weight, bias):
    idx = (x.astype(jnp.int32)
           .T.reshape(N_FIELDS, NW, ROWS_W)
           .transpose(1, 0, 2)
           .reshape(NW, IDX_W))
    table = _tc_squeeze(fc_weight.T)
    bias_b = jnp.broadcast_to(bias.astype(jnp.float32), (L,))
    out = _sc_embed_sum(idx, table, bias_b)
    return out.reshape(BATCH, 1)


# speedup vs baseline: 5.4504x; 1.0334x over previous
import functools

import jax
import jax.numpy as jnp
from jax import lax
from jax.experimental import pallas as pl
from jax.experimental.pallas import tpu as pltpu
from jax.experimental.pallas import tpu_sc as plsc

NUM_ROWS = 2600000
BATCH = 16384
N_FIELDS = 26

NC = 2
NS = 16
L = 16
NW = NC * NS
ROWS_W = BATCH // NW
IDX_W = ROWS_W * N_FIELDS
CHUNK = 128
NCHUNK = IDX_W // CHUNK

_mesh = plsc.VectorSubcoreMesh(core_axis_name="c", subcore_axis_name="s")

_SQ_BLK = 1048576
_SQ_GRID = -(-NUM_ROWS // _SQ_BLK)


def _tc_squeeze_body(w_ref, o_ref):
    o_ref[...] = w_ref[0, :]


_tc_squeeze = pl.pallas_call(
    _tc_squeeze_body,
    out_shape=jax.ShapeDtypeStruct((NUM_ROWS,), jnp.float32),
    grid=(_SQ_GRID,),
    in_specs=[pl.BlockSpec((1, _SQ_BLK), lambda i: (0, i))],
    out_specs=pl.BlockSpec((_SQ_BLK,), lambda i: (i,)),
)


@functools.partial(
    pl.kernel,
    out_type=jax.ShapeDtypeStruct((BATCH,), jnp.float32),
    mesh=_mesh,
    scratch_types=[
        pltpu.VMEM((IDX_W,), jnp.int32),
        pltpu.VMEM((IDX_W,), jnp.float32),
        pltpu.VMEM((ROWS_W,), jnp.float32),
        pltpu.VMEM((L,), jnp.float32),
        pltpu.SemaphoreType.DMA,
    ],
)
def _sc_embed_sum(idx_hbm, table_hbm, bias_hbm, out_hbm,
                  idx_v, vals_v, out_v, bias_v, sem):
    wid = lax.axis_index("s") * NC + lax.axis_index("c")
    base = wid * ROWS_W

    pltpu.sync_copy(bias_hbm, bias_v)
    pltpu.sync_copy(idx_hbm.at[wid], idx_v)

    @pl.loop(0, NCHUNK)
    def _fire(ch):
        pltpu.async_copy(table_hbm.at[idx_v.at[pl.ds(ch * CHUNK, CHUNK)]],
                         vals_v.at[pl.ds(ch * CHUNK, CHUNK)], sem)

    pltpu.make_async_copy(table_hbm.at[pl.ds(0, IDX_W)], vals_v, sem).wait()

    bias_vec = bias_v[...]

    @pl.loop(0, ROWS_W // L)
    def _reduce(j):
        acc = bias_vec
        for f in range(N_FIELDS):
            acc = acc + vals_v[pl.ds(f * ROWS_W + j * L, L)]
        out_v[pl.ds(j * L, L)] = acc

    pltpu.sync_copy(out_v, out_hbm.at[pl.ds(base, ROWS_W)])


def kernel(x, fc_weight, bias):
    idx = (x.astype(jnp.int32)
           .T.reshape(N_FIELDS, NW, ROWS_W)
           .transpose(1, 0, 2)
           .reshape(NW, IDX_W))
    table = _tc_squeeze(fc_weight.T)
    bias_b = jnp.broadcast_to(bias.astype(jnp.float32), (L,))
    out = _sc_embed_sum(idx, table, bias_b)
    return out.reshape(BATCH, 1)


# speedup vs baseline: 5.5196x; 1.0127x over previous
import functools

import jax
import jax.numpy as jnp
from jax import lax
from jax.experimental import pallas as pl
from jax.experimental.pallas import tpu as pltpu
from jax.experimental.pallas import tpu_sc as plsc

NUM_ROWS = 2600000
BATCH = 16384
N_FIELDS = 26

NC = 2
NS = 16
L = 16
NW = NC * NS
ROWS_W = BATCH // NW
IDX_W = ROWS_W * N_FIELDS
CHUNK = 128
NCHUNK = IDX_W // CHUNK

_mesh = plsc.VectorSubcoreMesh(core_axis_name="c", subcore_axis_name="s")

_SQ_BLK = 1310720
_SQ_GRID = -(-NUM_ROWS // _SQ_BLK)


def _tc_squeeze_body(w_ref, o_ref):
    o_ref[...] = w_ref[0, :]


_tc_squeeze = pl.pallas_call(
    _tc_squeeze_body,
    out_shape=jax.ShapeDtypeStruct((NUM_ROWS,), jnp.float32),
    grid=(_SQ_GRID,),
    in_specs=[pl.BlockSpec((1, _SQ_BLK), lambda i: (0, i))],
    out_specs=pl.BlockSpec((_SQ_BLK,), lambda i: (i,)),
)


@functools.partial(
    pl.kernel,
    out_type=jax.ShapeDtypeStruct((BATCH,), jnp.float32),
    mesh=_mesh,
    scratch_types=[
        pltpu.VMEM((IDX_W,), jnp.int32),
        pltpu.VMEM((IDX_W,), jnp.float32),
        pltpu.VMEM((ROWS_W,), jnp.float32),
        pltpu.VMEM((L,), jnp.float32),
        pltpu.SemaphoreType.DMA,
    ],
)
def _sc_embed_sum(idx_hbm, table_hbm, bias_hbm, out_hbm,
                  idx_v, vals_v, out_v, bias_v, sem):
    wid = lax.axis_index("s") * NC + lax.axis_index("c")
    base = wid * ROWS_W

    pltpu.sync_copy(bias_hbm, bias_v)
    pltpu.sync_copy(idx_hbm.at[wid], idx_v)

    @pl.loop(0, NCHUNK)
    def _fire(ch):
        pltpu.async_copy(table_hbm.at[idx_v.at[pl.ds(ch * CHUNK, CHUNK)]],
                         vals_v.at[pl.ds(ch * CHUNK, CHUNK)], sem)

    pltpu.make_async_copy(table_hbm.at[pl.ds(0, IDX_W)], vals_v, sem).wait()

    bias_vec = bias_v[...]

    @pl.loop(0, ROWS_W // L)
    def _reduce(j):
        acc = bias_vec
        for f in range(N_FIELDS):
            acc = acc + vals_v[pl.ds(f * ROWS_W + j * L, L)]
        out_v[pl.ds(j * L, L)] = acc

    pltpu.sync_copy(out_v, out_hbm.at[pl.ds(base, ROWS_W)])


def kernel(x, fc_weight, bias):
    idx = (x.astype(jnp.int32)
           .T.reshape(N_FIELDS, NW, ROWS_W)
           .transpose(1, 0, 2)
           .reshape(NW, IDX_W))
    table = _tc_squeeze(fc_weight.T)
    bias_b = jnp.broadcast_to(bias.astype(jnp.float32), (L,))
    out = _sc_embed_sum(idx, table, bias_b)
    return out.reshape(BATCH, 1)
